# baseline (speedup 1.0000x reference)
import functools

import jax
import jax.numpy as jnp
from jax import lax
from jax.experimental import pallas as pl
from jax.experimental.pallas import tpu as pltpu
from jax.experimental.pallas import tpu_sc as plsc

F32 = jnp.float32
I32 = jnp.int32

N = 50000
NP = 50176
E = 800000
EP = 802816
ER = EP // 64
F = 2048
M = 256
D = 128
BUCKET = 12544
ACC_ROWS = BUCKET + 128

_mesh = plsc.VectorSubcoreMesh(core_axis_name="c", subcore_axis_name="s")


EC = 64
NCHUNK = 784


@functools.partial(
    pl.kernel,
    out_type=jax.ShapeDtypeStruct((NP, D), F32),
    mesh=_mesh,
    scratch_types=[
        pltpu.VMEM_SHARED((ACC_ROWS, D), F32),
        pltpu.VMEM((2, 1, EC), I32),
        pltpu.VMEM((2, 1, EC), I32),
        pltpu.VMEM((2, EC, D), F32),
        pltpu.SemaphoreType.DMA,
        pltpu.SemaphoreType.DMA,
    ],
)
def _sc_agg(h, src2, dst2, zeros, out, acc, src_v, dst_v, rows_v, gsem, ssem):
    c = lax.axis_index("c")
    sid = lax.axis_index("s")
    lanes = lax.broadcasted_iota(I32, (16,), 0)
    dummy = BUCKET + (sid % 8) * 16 + lanes

    def load_remap_gather(ci, b, lo):
        row0 = sid * NCHUNK + ci
        pltpu.sync_copy(src2.at[pl.ds(row0, 1)], src_v.at[b])
        pltpu.sync_copy(dst2.at[pl.ds(row0, 1)], dst_v.at[b])
        dr = dst_v.at[b].at[0]
        for i in range(EC // 16):
            d = dr[pl.ds(i * 16, 16)]
            m = (d >= lo) & (d < lo + BUCKET)
            dr[pl.ds(i * 16, 16)] = jnp.where(m, d - lo, dummy)
        pltpu.async_copy(h.at[src_v.at[b].at[0]], rows_v.at[b], gsem)

    def wait_gather(b):
        pltpu.make_async_copy(h.at[src_v.at[b].at[0]], rows_v.at[b], gsem).wait()

    def start_scatter(b):
        pltpu.async_copy(rows_v.at[b], acc.at[dst_v.at[b].at[0]], ssem, add=True)

    def wait_scatter(b):
        pltpu.make_async_copy(rows_v.at[b], acc.at[dst_v.at[b].at[0]], ssem).wait()

    for p in range(2):
        lo = (2 * c + p) * BUCKET
        pltpu.sync_copy(zeros.at[pl.ds(sid * 792, 792)],
                        acc.at[pl.ds(sid * 792, 792)])
        plsc.subcore_barrier()

        load_remap_gather(0, 0, lo)

        def pair(g, carry):
            for b in (0, 1):
                ci = g * 2 + b
                nb = 1 - b

                @pl.when(ci >= 1)
                def _():
                    wait_scatter(nb)

                @pl.when(ci + 1 < NCHUNK)
                def _():
                    load_remap_gather(ci + 1, nb, lo)

                wait_gather(b)
                start_scatter(b)
            return carry

        lax.fori_loop(0, NCHUNK // 2, pair, 0)
        wait_scatter(1)
        plsc.subcore_barrier()
        pltpu.sync_copy(acc.at[pl.ds(sid * 784, 784)],
                        out.at[pl.ds(lo + sid * 784, 784)])
        plsc.subcore_barrier()


@functools.partial(
    pl.kernel,
    out_type=(jax.ShapeDtypeStruct((2 * F, D), F32),
              jax.ShapeDtypeStruct((2 * F, D), F32)),
    mesh=_mesh,
    scratch_types=[
        pltpu.VMEM_SHARED((2176, D), F32),
        pltpu.VMEM_SHARED((2176, D), F32),
        pltpu.VMEM((128, D), F32),
        pltpu.VMEM((128, D), F32),
        pltpu.VMEM((128,), I32),
        pltpu.VMEM((32,), I32),
    ],
)
def _sc_pool(h, batch1, ones_in, zeros, sums, cnts,
             acc_s, acc_c, rows_v, ones_v, idx_v, idxt_v):
    c = lax.axis_index("c")
    sid = lax.axis_index("s")
    wid = sid * 2 + c
    pltpu.sync_copy(zeros.at[pl.ds(sid * 136, 136)],
                    acc_s.at[pl.ds(sid * 136, 136)])
    pltpu.sync_copy(zeros.at[pl.ds(sid * 136, 136)],
                    acc_c.at[pl.ds(sid * 136, 136)])
    pltpu.sync_copy(ones_in, ones_v)
    plsc.subcore_barrier()
    base = wid * 1568
    for k in range(12):
        n0 = base + k * 128
        pltpu.sync_copy(batch1.at[pl.ds(n0, 128)], idx_v)
        pltpu.sync_copy(h.at[pl.ds(n0, 128)], rows_v)
        pltpu.sync_copy(rows_v, acc_s.at[idx_v], add=True)
        pltpu.sync_copy(ones_v, acc_c.at[idx_v], add=True)
    n0 = base + 1536
    pltpu.sync_copy(batch1.at[pl.ds(n0, 32)], idxt_v)
    pltpu.sync_copy(h.at[pl.ds(n0, 32)], rows_v.at[pl.ds(0, 32)])
    pltpu.sync_copy(rows_v.at[pl.ds(0, 32)], acc_s.at[idxt_v], add=True)
    pltpu.sync_copy(ones_v.at[pl.ds(0, 32)], acc_c.at[idxt_v], add=True)
    plsc.subcore_barrier()
    pltpu.sync_copy(acc_s.at[pl.ds(sid * 128, 128)],
                    sums.at[pl.ds(c * F + sid * 128, 128)])
    pltpu.sync_copy(acc_c.at[pl.ds(sid * 128, 128)],
                    cnts.at[pl.ds(c * F + sid * 128, 128)])


BM = 3584


def _proj_body(x_ref, w1_ref, o_ref):
    o_ref[...] = jnp.dot(x_ref[...], w1_ref[...], preferred_element_type=F32)


def _mlp0_body(y_ref, a_ref, b1_ref, w2_ref, b2_ref, s_ref, t_ref, o_ref):
    z = jnp.maximum(y_ref[...] + a_ref[...] + b1_ref[...], 0.0)
    z = jnp.dot(z, w2_ref[...], preferred_element_type=F32) + b2_ref[...]
    o_ref[...] = jnp.maximum(z * s_ref[...] + t_ref[...], 0.0)


def _mlp_body(x_ref, a_ref, w1_ref, b1_ref, w2_ref, b2_ref,
              s_ref, t_ref, o_ref):
    z = x_ref[...] + a_ref[...]
    z = jnp.dot(z, w1_ref[...], preferred_element_type=F32) + b1_ref[...]
    z = jnp.maximum(z, 0.0)
    z = jnp.dot(z, w2_ref[...], preferred_element_type=F32) + b2_ref[...]
    o_ref[...] = jnp.maximum(z * s_ref[...] + t_ref[...], 0.0)


def _row_spec(w):
    return pl.BlockSpec((BM, w), lambda i: (i, 0))


def _full_spec(shape):
    return pl.BlockSpec(shape, lambda i: tuple(0 for _ in shape))


def _proj(xp, w1):
    return pl.pallas_call(
        _proj_body,
        grid=(NP // BM,),
        in_specs=[_row_spec(16), _full_spec((16, D))],
        out_specs=_row_spec(D),
        out_shape=jax.ShapeDtypeStruct((NP, D), F32),
    )(xp, w1)


def _mlp0(y, a, b1, w2, b2, s, t):
    return pl.pallas_call(
        _mlp0_body,
        grid=(NP // BM,),
        in_specs=[_row_spec(D), _row_spec(D),
                  _full_spec((1, D)),
                  _full_spec((D, D)), _full_spec((1, D)),
                  _full_spec((1, D)), _full_spec((1, D))],
        out_specs=_row_spec(D),
        out_shape=jax.ShapeDtypeStruct((NP, D), F32),
    )(y, a, b1, w2, b2, s, t)


def _mlp(h, a, w1, b1, w2, b2, s, t):
    return pl.pallas_call(
        _mlp_body,
        grid=(NP // BM,),
        in_specs=[_row_spec(D), _row_spec(D),
                  _full_spec((D, D)), _full_spec((1, D)),
                  _full_spec((D, D)), _full_spec((1, D)),
                  _full_spec((1, D)), _full_spec((1, D))],
        out_specs=_row_spec(D),
        out_shape=jax.ShapeDtypeStruct((NP, D), F32),
    )(h, a, w1, b1, w2, b2, s, t)


def _head_body(sums_ref, cnts_ref, mol_ref, rw1_ref, rb1_ref, rw2_ref, rb2_ref,
               ew1_ref, eb1_ref, ew2_ref, eb2_ref, ew3_ref, eb3_ref, o_ref):
    s = sums_ref[0] + sums_ref[1]
    cnt = cnts_ref[0] + cnts_ref[1]
    femb = s / jnp.maximum(cnt, 1.0)
    r = jnp.dot(femb, rw1_ref[...], preferred_element_type=F32) + rb1_ref[...]
    r = jnp.maximum(r, 0.0)
    logits = jnp.dot(r, rw2_ref[...], preferred_element_type=F32) + rb2_ref[...]
    logits = logits - jnp.max(logits, axis=1, keepdims=True)
    ew = jnp.exp(logits)
    w = ew / jnp.sum(ew, axis=1, keepdims=True)
    wsum = jnp.zeros((F, 1), F32)
    for e in range(4):
        te = jnp.dot(femb, ew1_ref[e], preferred_element_type=F32) + eb1_ref[e]
        te = jnp.maximum(te, 0.0)
        te = jnp.dot(te, ew2_ref[e], preferred_element_type=F32) + eb2_ref[e]
        te = jnp.maximum(te, 0.0)
        t3 = jnp.dot(te, ew3_ref[e], preferred_element_type=F32) + eb3_ref[e]
        wsum = wsum + w[:, e:e + 1] * t3[:, 0:1]
    onehot = (mol_ref[...] == lax.broadcasted_iota(I32, (F, M), 1)).astype(F32)
    wb = jnp.broadcast_to(wsum, (F, D))
    msum = lax.dot_general(onehot, wb, (((0,), (0,)), ((), ())),
                           preferred_element_type=F32)
    mcnt = lax.dot_general(onehot, jnp.ones((F, D), F32),
                           (((0,), (0,)), ((), ())),
                           preferred_element_type=F32)
    o_ref[...] = jnp.where(mcnt > 0, msum / jnp.maximum(mcnt, 1.0), 0.0)


def _head(sums, cnts, mol2, rw1, rb1, rw2, rb2, ew1, eb1, ew2, eb2, ew3, eb3):
    return pl.pallas_call(
        _head_body,
        out_shape=jax.ShapeDtypeStruct((M, D), F32),
    )(sums, cnts, mol2, rw1, rb1, rw2, rb2, ew1, eb1, ew2, eb2, ew3, eb3)


def _pad_cols(a, w):
    return jnp.pad(a, ((0, 0), (0, w - a.shape[1])))


def kernel(x, edge_index, batch, mol_idx, params):
    src = edge_index[0].astype(I32)
    dst = edge_index[1].astype(I32)
    padi = jnp.arange(EP - E, dtype=I32)
    src2 = jnp.concatenate([src, padi % N]).reshape(ER, 64)
    dst2 = jnp.concatenate([dst, N + padi % (NP - N)]).reshape(ER, 64)
    xp = jnp.zeros((NP, 16), F32).at[:N, :9].set(x)
    batchp = jnp.concatenate(
        [batch.astype(I32), F + jnp.arange(NP - N, dtype=I32) % 64])
    zeros_big = jnp.zeros((ACC_ROWS, D), F32)
    ones128 = jnp.ones((128, D), F32)

    gin = params['gin']
    sts = []
    for layer in gin:
        sc = layer['gamma'] * lax.rsqrt(layer['var'] + 1e-5)
        sh = layer['beta'] - layer['mean'] * sc
        sts.append((sc.reshape(1, D), sh.reshape(1, D)))

    w1p = jnp.zeros((16, D), F32).at[:9, :].set(gin[0]['w1'])
    y0 = _proj(xp, w1p)
    agg0 = _sc_agg(y0, src2, dst2, zeros_big)
    h = _mlp0(y0, agg0, gin[0]['b1'].reshape(1, D),
              gin[0]['w2'], gin[0]['b2'].reshape(1, D), *sts[0])

    for li in (1, 2):
        agg = _sc_agg(h, src2, dst2, zeros_big)
        h = _mlp(h, agg, gin[li]['w1'], gin[li]['b1'].reshape(1, D),
                 gin[li]['w2'], gin[li]['b2'].reshape(1, D), *sts[li])

    sums, cnts = _sc_pool(h, batchp, ones128, zeros_big)
    sums = sums.reshape(2, F, D)
    cnts = cnts.reshape(2, F, D)

    rw1p = _pad_cols(params['rw1'], D)
    rb1p = _pad_cols(params['rb1'].reshape(1, -1), D)
    rw2p = jnp.zeros((D, D), F32).at[:64, :4].set(params['rw2'])
    rb2p = jnp.full((1, D), -1e9, F32).at[0, :4].set(params['rb2'])
    ew1 = jnp.stack([e['w1'] for e in params['experts']])
    eb1 = jnp.stack([e['b1'].reshape(1, D) for e in params['experts']])
    ew2 = jnp.stack([_pad_cols(e['w2'], D) for e in params['experts']])
    eb2 = jnp.stack([_pad_cols(e['b2'].reshape(1, -1), D)
                     for e in params['experts']])
    ew3 = jnp.stack([jnp.zeros((D, D), F32).at[:64, 0:1].set(e['w3'])
                     for e in params['experts']])
    eb3 = jnp.stack([jnp.zeros((1, D), F32).at[0, 0].set(e['b3'][0])
                     for e in params['experts']])
    mol2 = mol_idx.astype(I32).reshape(F, 1)

    out = _head(sums, cnts, mol2, rw1p, rb1p, rw2p, rb2p,
                ew1, eb1, ew2, eb2, ew3, eb3)
    return out[:, 0]


# speedup vs baseline: 1.2225x; 1.2225x over previous
import functools

import jax
import jax.numpy as jnp
from jax import lax
from jax.experimental import pallas as pl
from jax.experimental.pallas import tpu as pltpu
from jax.experimental.pallas import tpu_sc as plsc

F32 = jnp.float32
I32 = jnp.int32

N = 50000
NP = 50176
E = 800000
EP = 802816
ER = EP // 128
F = 2048
M = 256
D = 128
BUCKET = 12544
ACC_ROWS = BUCKET + 128

_mesh = plsc.VectorSubcoreMesh(core_axis_name="c", subcore_axis_name="s")


def _splat_lane(vec, k):
    idx = jnp.full((16,), k, I32)
    return vec.at[idx].get(mode="promise_in_bounds")


@functools.partial(
    pl.kernel,
    out_type=jax.ShapeDtypeStruct((NP, D), F32),
    mesh=_mesh,
    scratch_types=[
        pltpu.VMEM_SHARED((ACC_ROWS, D), F32),
        pltpu.VMEM((8, 128), I32),
        pltpu.VMEM((8, 128), I32),
        pltpu.VMEM((128, D), F32),
    ],
)
def _sc_agg(h, src2, dst2, zeros, out, acc, src_v, dst_v, rows_v):
    c = lax.axis_index("c")
    sid = lax.axis_index("s")
    lanes = lax.broadcasted_iota(I32, (16,), 0)
    dummy = BUCKET + (sid % 8) * 16 + lanes
    for p in range(2):
        lo = (2 * c + p) * BUCKET
        pltpu.sync_copy(zeros.at[pl.ds(sid * 792, 792)],
                        acc.at[pl.ds(sid * 792, 792)])
        plsc.subcore_barrier()

        def chunk(g, carry):
            row0 = sid * 392 + g * 8
            pltpu.sync_copy(src2.at[pl.ds(row0, 8)], src_v)
            pltpu.sync_copy(dst2.at[pl.ds(row0, 8)], dst_v)
            for j in range(8):
                dr = dst_v.at[j]
                for i in range(8):
                    d = dr[pl.ds(i * 16, 16)]
                    m = (d >= lo) & (d < lo + BUCKET)
                    dr[pl.ds(i * 16, 16)] = jnp.where(m, d - lo, dummy)
                pltpu.sync_copy(h.at[src_v.at[j]], rows_v)
                pltpu.sync_copy(rows_v, acc.at[dst_v.at[j]], add=True)
            return carry

        lax.fori_loop(0, 49, chunk, 0)
        plsc.subcore_barrier()
        pltpu.sync_copy(acc.at[pl.ds(sid * 784, 784)],
                        out.at[pl.ds(lo + sid * 784, 784)])
        plsc.subcore_barrier()


@functools.partial(
    pl.kernel,
    out_type=(jax.ShapeDtypeStruct((2 * F, D), F32),
              jax.ShapeDtypeStruct((2 * F, D), F32)),
    mesh=_mesh,
    scratch_types=[
        pltpu.VMEM_SHARED((2176, D), F32),
        pltpu.VMEM_SHARED((2176, D), F32),
        pltpu.VMEM((128, D), F32),
        pltpu.VMEM((128, D), F32),
        pltpu.VMEM((128,), I32),
        pltpu.VMEM((32,), I32),
    ],
)
def _sc_pool(h, batch1, ones_in, zeros, sums, cnts,
             acc_s, acc_c, rows_v, ones_v, idx_v, idxt_v):
    c = lax.axis_index("c")
    sid = lax.axis_index("s")
    wid = sid * 2 + c
    pltpu.sync_copy(zeros.at[pl.ds(sid * 136, 136)],
                    acc_s.at[pl.ds(sid * 136, 136)])
    pltpu.sync_copy(zeros.at[pl.ds(sid * 136, 136)],
                    acc_c.at[pl.ds(sid * 136, 136)])
    pltpu.sync_copy(ones_in, ones_v)
    plsc.subcore_barrier()
    base = wid * 1568
    for k in range(12):
        n0 = base + k * 128
        pltpu.sync_copy(batch1.at[pl.ds(n0, 128)], idx_v)
        pltpu.sync_copy(h.at[pl.ds(n0, 128)], rows_v)
        pltpu.sync_copy(rows_v, acc_s.at[idx_v], add=True)
        pltpu.sync_copy(ones_v, acc_c.at[idx_v], add=True)
    n0 = base + 1536
    pltpu.sync_copy(batch1.at[pl.ds(n0, 32)], idxt_v)
    pltpu.sync_copy(h.at[pl.ds(n0, 32)], rows_v.at[pl.ds(0, 32)])
    pltpu.sync_copy(rows_v.at[pl.ds(0, 32)], acc_s.at[idxt_v], add=True)
    pltpu.sync_copy(ones_v.at[pl.ds(0, 32)], acc_c.at[idxt_v], add=True)
    plsc.subcore_barrier()
    pltpu.sync_copy(acc_s.at[pl.ds(sid * 128, 128)],
                    sums.at[pl.ds(c * F + sid * 128, 128)])
    pltpu.sync_copy(acc_c.at[pl.ds(sid * 128, 128)],
                    cnts.at[pl.ds(c * F + sid * 128, 128)])


BM = 3584


def _proj_body(x_ref, w1_ref, o_ref):
    o_ref[...] = jnp.dot(x_ref[...], w1_ref[...], preferred_element_type=F32)


def _mlp0_body(y_ref, a_ref, b1_ref, w2_ref, b2_ref, s_ref, t_ref, o_ref):
    z = jnp.maximum(y_ref[...] + a_ref[...] + b1_ref[...], 0.0)
    z = jnp.dot(z, w2_ref[...], preferred_element_type=F32) + b2_ref[...]
    o_ref[...] = jnp.maximum(z * s_ref[...] + t_ref[...], 0.0)


def _mlp_body(x_ref, a_ref, w1_ref, b1_ref, w2_ref, b2_ref,
              s_ref, t_ref, o_ref):
    z = x_ref[...] + a_ref[...]
    z = jnp.dot(z, w1_ref[...], preferred_element_type=F32) + b1_ref[...]
    z = jnp.maximum(z, 0.0)
    z = jnp.dot(z, w2_ref[...], preferred_element_type=F32) + b2_ref[...]
    o_ref[...] = jnp.maximum(z * s_ref[...] + t_ref[...], 0.0)


def _row_spec(w):
    return pl.BlockSpec((BM, w), lambda i: (i, 0))


def _full_spec(shape):
    return pl.BlockSpec(shape, lambda i: tuple(0 for _ in shape))


def _proj(xp, w1):
    return pl.pallas_call(
        _proj_body,
        grid=(NP // BM,),
        in_specs=[_row_spec(16), _full_spec((16, D))],
        out_specs=_row_spec(D),
        out_shape=jax.ShapeDtypeStruct((NP, D), F32),
    )(xp, w1)


def _mlp0(y, a, b1, w2, b2, s, t):
    return pl.pallas_call(
        _mlp0_body,
        grid=(NP // BM,),
        in_specs=[_row_spec(D), _row_spec(D),
                  _full_spec((1, D)),
                  _full_spec((D, D)), _full_spec((1, D)),
                  _full_spec((1, D)), _full_spec((1, D))],
        out_specs=_row_spec(D),
        out_shape=jax.ShapeDtypeStruct((NP, D), F32),
    )(y, a, b1, w2, b2, s, t)


def _mlp(h, a, w1, b1, w2, b2, s, t):
    return pl.pallas_call(
        _mlp_body,
        grid=(NP // BM,),
        in_specs=[_row_spec(D), _row_spec(D),
                  _full_spec((D, D)), _full_spec((1, D)),
                  _full_spec((D, D)), _full_spec((1, D)),
                  _full_spec((1, D)), _full_spec((1, D))],
        out_specs=_row_spec(D),
        out_shape=jax.ShapeDtypeStruct((NP, D), F32),
    )(h, a, w1, b1, w2, b2, s, t)


def _head_body(sums_ref, cnts_ref, mol_ref, rw1_ref, rb1_ref, rw2_ref, rb2_ref,
               ew1_ref, eb1_ref, ew2_ref, eb2_ref, ew3_ref, eb3_ref, o_ref):
    s = sums_ref[0] + sums_ref[1]
    cnt = cnts_ref[0] + cnts_ref[1]
    femb = s / jnp.maximum(cnt, 1.0)
    r = jnp.dot(femb, rw1_ref[...], preferred_element_type=F32) + rb1_ref[...]
    r = jnp.maximum(r, 0.0)
    logits = jnp.dot(r, rw2_ref[...], preferred_element_type=F32) + rb2_ref[...]
    logits = logits - jnp.max(logits, axis=1, keepdims=True)
    ew = jnp.exp(logits)
    w = ew / jnp.sum(ew, axis=1, keepdims=True)
    wsum = jnp.zeros((F, 1), F32)
    for e in range(4):
        te = jnp.dot(femb, ew1_ref[e], preferred_element_type=F32) + eb1_ref[e]
        te = jnp.maximum(te, 0.0)
        te = jnp.dot(te, ew2_ref[e], preferred_element_type=F32) + eb2_ref[e]
        te = jnp.maximum(te, 0.0)
        t3 = jnp.dot(te, ew3_ref[e], preferred_element_type=F32) + eb3_ref[e]
        wsum = wsum + w[:, e:e + 1] * t3[:, 0:1]
    onehot = (mol_ref[...] == lax.broadcasted_iota(I32, (F, M), 1)).astype(F32)
    wb = jnp.broadcast_to(wsum, (F, D))
    msum = lax.dot_general(onehot, wb, (((0,), (0,)), ((), ())),
                           preferred_element_type=F32)
    mcnt = lax.dot_general(onehot, jnp.ones((F, D), F32),
                           (((0,), (0,)), ((), ())),
                           preferred_element_type=F32)
    o_ref[...] = jnp.where(mcnt > 0, msum / jnp.maximum(mcnt, 1.0), 0.0)


def _head(sums, cnts, mol2, rw1, rb1, rw2, rb2, ew1, eb1, ew2, eb2, ew3, eb3):
    return pl.pallas_call(
        _head_body,
        out_shape=jax.ShapeDtypeStruct((M, D), F32),
    )(sums, cnts, mol2, rw1, rb1, rw2, rb2, ew1, eb1, ew2, eb2, ew3, eb3)


def _pad_cols(a, w):
    return jnp.pad(a, ((0, 0), (0, w - a.shape[1])))


def kernel(x, edge_index, batch, mol_idx, params):
    src = edge_index[0].astype(I32)
    dst = edge_index[1].astype(I32)
    padi = jnp.arange(EP - E, dtype=I32)
    src2 = jnp.concatenate([src, padi % N]).reshape(ER, 128)
    dst2 = jnp.concatenate([dst, N + padi % (NP - N)]).reshape(ER, 128)
    xp = jnp.zeros((NP, 16), F32).at[:N, :9].set(x)
    batchp = jnp.concatenate(
        [batch.astype(I32), F + jnp.arange(NP - N, dtype=I32) % 64])
    zeros_big = jnp.zeros((ACC_ROWS, D), F32)
    ones128 = jnp.ones((128, D), F32)

    gin = params['gin']
    sts = []
    for layer in gin:
        sc = layer['gamma'] * lax.rsqrt(layer['var'] + 1e-5)
        sh = layer['beta'] - layer['mean'] * sc
        sts.append((sc.reshape(1, D), sh.reshape(1, D)))

    w1p = jnp.zeros((16, D), F32).at[:9, :].set(gin[0]['w1'])
    y0 = _proj(xp, w1p)
    agg0 = _sc_agg(y0, src2, dst2, zeros_big)
    h = _mlp0(y0, agg0, gin[0]['b1'].reshape(1, D),
              gin[0]['w2'], gin[0]['b2'].reshape(1, D), *sts[0])

    for li in (1, 2):
        agg = _sc_agg(h, src2, dst2, zeros_big)
        h = _mlp(h, agg, gin[li]['w1'], gin[li]['b1'].reshape(1, D),
                 gin[li]['w2'], gin[li]['b2'].reshape(1, D), *sts[li])

    sums, cnts = _sc_pool(h, batchp, ones128, zeros_big)
    sums = sums.reshape(2, F, D)
    cnts = cnts.reshape(2, F, D)

    rw1p = _pad_cols(params['rw1'], D)
    rb1p = _pad_cols(params['rb1'].reshape(1, -1), D)
    rw2p = jnp.zeros((D, D), F32).at[:64, :4].set(params['rw2'])
    rb2p = jnp.full((1, D), -1e9, F32).at[0, :4].set(params['rb2'])
    ew1 = jnp.stack([e['w1'] for e in params['experts']])
    eb1 = jnp.stack([e['b1'].reshape(1, D) for e in params['experts']])
    ew2 = jnp.stack([_pad_cols(e['w2'], D) for e in params['experts']])
    eb2 = jnp.stack([_pad_cols(e['b2'].reshape(1, -1), D)
                     for e in params['experts']])
    ew3 = jnp.stack([jnp.zeros((D, D), F32).at[:64, 0:1].set(e['w3'])
                     for e in params['experts']])
    eb3 = jnp.stack([jnp.zeros((1, D), F32).at[0, 0].set(e['b3'][0])
                     for e in params['experts']])
    mol2 = mol_idx.astype(I32).reshape(F, 1)

    out = _head(sums, cnts, mol2, rw1p, rb1p, rw2p, rb2p,
                ew1, eb1, ew2, eb2, ew3, eb3)
    return out[:, 0]
